# baseline (speedup 1.0000x reference)
import functools

import jax
import jax.numpy as jnp
from jax import lax
from jax.experimental import pallas as pl
from jax.experimental.pallas import tpu as pltpu
from jax.experimental.pallas import tpu_sc as plsc

N = 10000
DF = 128
E = 320000
L1S = 128
L2S = 64
OUTS = 16
EPS = 1e-5

CHUNK = 128
NC = 2
NS = 16
NW = NC * NS
NCH_T = 80
NCHP = NW * NCH_T
EPAD = NCHP * CHUNK
NROWS = N + CHUNK
RPT = N // NS
CW = 16
NPAIR = NCH_T // 2

BR = 400
GRID = N // BR


def _sc_mesh():
    return plsc.VectorSubcoreMesh(core_axis_name="c", subcore_axis_name="s")


_SC_PARAMS = pltpu.CompilerParams(use_tc_tiling_on_sc=False)


def _agg_body_common(h_hbm, src_hbm, dst_hbm, zf_hbm, agg_s,
                     isrc, idst, rows_a, rows_b, sem_a, sem_b,
                     wid, with_cnt=None):
    ones_v, cnt_s = with_cnt if with_cnt is not None else (None, None)
    start = wid * NCH_T

    def stage(pair, slot):
        pltpu.sync_copy(src_hbm.at[pl.ds(start + 2 * pair, 2)], isrc.at[slot])
        pltpu.sync_copy(dst_hbm.at[pl.ds(start + 2 * pair, 2)], idst.at[slot])

    def fire(slot, b, rows, sem):
        pltpu.async_copy(h_hbm.at[isrc.at[slot, b]], rows, sem)

    def drain_scatter(slot, b, rows, sem):
        pltpu.make_async_copy(zf_hbm.at[pl.ds(0, CHUNK)], rows, sem).wait()
        pltpu.sync_copy(rows, agg_s.at[idst.at[slot, b]], add=True)
        if ones_v is not None:
            pltpu.sync_copy(ones_v, cnt_s.at[idst.at[slot, b]], add=True)

    stage(0, 0)
    fire(0, 0, rows_a, sem_a)
    fire(0, 1, rows_b, sem_b)

    def step(t2, carry):
        p0 = 2 * t2
        p1 = 2 * t2 + 1

        stage(p1, 1)
        drain_scatter(0, 0, rows_a, sem_a)
        fire(1, 0, rows_a, sem_a)
        drain_scatter(0, 1, rows_b, sem_b)
        fire(1, 1, rows_b, sem_b)

        @pl.when(p1 + 1 < NPAIR)
        def _():
            stage(p1 + 1, 0)

        drain_scatter(1, 0, rows_a, sem_a)

        @pl.when(p1 + 1 < NPAIR)
        def _():
            fire(0, 0, rows_a, sem_a)

        drain_scatter(1, 1, rows_b, sem_b)

        @pl.when(p1 + 1 < NPAIR)
        def _():
            fire(0, 1, rows_b, sem_b)

        return carry

    lax.fori_loop(0, NPAIR // 2, step, 0)


def _sc_agg_with_cnt(h, src2d, dst2d, zf, zc, ones16):
    f32 = jnp.float32

    @functools.partial(
        pl.kernel,
        mesh=_sc_mesh(),
        out_type=[
            jax.ShapeDtypeStruct((N, DF), f32),
            jax.ShapeDtypeStruct((N, DF), f32),
            jax.ShapeDtypeStruct((N, CW), f32),
            jax.ShapeDtypeStruct((N, CW), f32),
        ],
        scratch_types=[
            pltpu.VMEM_SHARED((NROWS, DF), f32),
            pltpu.VMEM_SHARED((NROWS, CW), f32),
            pltpu.VMEM((2, 2, CHUNK), jnp.int32),
            pltpu.VMEM((2, 2, CHUNK), jnp.int32),
            pltpu.VMEM((CHUNK, DF), f32),
            pltpu.VMEM((CHUNK, DF), f32),
            pltpu.VMEM((CHUNK, CW), f32),
            pltpu.SemaphoreType.DMA,
            pltpu.SemaphoreType.DMA,
        ],
        compiler_params=_SC_PARAMS,
    )
    def k(h_hbm, src_hbm, dst_hbm, zf_hbm, zc_hbm, ones_hbm,
          a0_out, a1_out, c0_out, c1_out,
          agg_s, cnt_s, isrc, idst, rows_a, rows_b, ones_v, sem_a, sem_b):
        cid = lax.axis_index("c")
        sid = lax.axis_index("s")
        wid = sid * NC + cid
        r0 = sid * RPT
        pltpu.sync_copy(zf_hbm.at[pl.ds(r0, RPT)], agg_s.at[pl.ds(r0, RPT)])
        pltpu.sync_copy(zc_hbm.at[pl.ds(r0, RPT)], cnt_s.at[pl.ds(r0, RPT)])
        pltpu.sync_copy(ones_hbm, ones_v)

        @pl.when(sid == NS - 1)
        def _():
            pltpu.sync_copy(zf_hbm.at[pl.ds(N, NROWS - N)],
                            agg_s.at[pl.ds(N, NROWS - N)])
            pltpu.sync_copy(zc_hbm.at[pl.ds(N, NROWS - N)],
                            cnt_s.at[pl.ds(N, NROWS - N)])

        plsc.subcore_barrier()
        _agg_body_common(h_hbm, src_hbm, dst_hbm, zf_hbm, agg_s,
                         isrc, idst, rows_a, rows_b, sem_a, sem_b,
                         wid, with_cnt=(ones_v, cnt_s))
        plsc.subcore_barrier()

        @pl.when(cid == 0)
        def _():
            pltpu.sync_copy(agg_s.at[pl.ds(r0, RPT)], a0_out.at[pl.ds(r0, RPT)])
            pltpu.sync_copy(cnt_s.at[pl.ds(r0, RPT)], c0_out.at[pl.ds(r0, RPT)])

        @pl.when(cid == 1)
        def _():
            pltpu.sync_copy(agg_s.at[pl.ds(r0, RPT)], a1_out.at[pl.ds(r0, RPT)])
            pltpu.sync_copy(cnt_s.at[pl.ds(r0, RPT)], c1_out.at[pl.ds(r0, RPT)])

    return k(h, src2d, dst2d, zf, zc, ones16)


def _sc_agg(h, src2d, dst2d, zf):
    f32 = jnp.float32

    @functools.partial(
        pl.kernel,
        mesh=_sc_mesh(),
        out_type=[
            jax.ShapeDtypeStruct((N, DF), f32),
            jax.ShapeDtypeStruct((N, DF), f32),
        ],
        scratch_types=[
            pltpu.VMEM_SHARED((NROWS, DF), f32),
            pltpu.VMEM((2, 2, CHUNK), jnp.int32),
            pltpu.VMEM((2, 2, CHUNK), jnp.int32),
            pltpu.VMEM((CHUNK, DF), f32),
            pltpu.VMEM((CHUNK, DF), f32),
            pltpu.SemaphoreType.DMA,
            pltpu.SemaphoreType.DMA,
        ],
        compiler_params=_SC_PARAMS,
    )
    def k(h_hbm, src_hbm, dst_hbm, zf_hbm,
          a0_out, a1_out,
          agg_s, isrc, idst, rows_a, rows_b, sem_a, sem_b):
        cid = lax.axis_index("c")
        sid = lax.axis_index("s")
        wid = sid * NC + cid
        r0 = sid * RPT
        pltpu.sync_copy(zf_hbm.at[pl.ds(r0, RPT)], agg_s.at[pl.ds(r0, RPT)])

        @pl.when(sid == NS - 1)
        def _():
            pltpu.sync_copy(zf_hbm.at[pl.ds(N, NROWS - N)],
                            agg_s.at[pl.ds(N, NROWS - N)])

        plsc.subcore_barrier()
        _agg_body_common(h_hbm, src_hbm, dst_hbm, zf_hbm, agg_s,
                         isrc, idst, rows_a, rows_b, sem_a, sem_b, wid)
        plsc.subcore_barrier()

        @pl.when(cid == 0)
        def _():
            pltpu.sync_copy(agg_s.at[pl.ds(r0, RPT)], a0_out.at[pl.ds(r0, RPT)])

        @pl.when(cid == 1)
        def _():
            pltpu.sync_copy(agg_s.at[pl.ds(r0, RPT)], a1_out.at[pl.ds(r0, RPT)])

    return k(h, src2d, dst2d, zf)


def _mm_t(a, w):
    return lax.dot_general(a, w, (((1,), (1,)), ((), ())),
                           preferred_element_type=jnp.float32)


def _bn_apply(x, acc, g, b):
    mean = acc[0:1, :] * (1.0 / N)
    var = acc[1:2, :] * (1.0 / N) - mean * mean
    return (x - mean) * lax.rsqrt(var + EPS) * g + b


def _pre(x, g, b):
    def body(x_ref, g_ref, b_ref, h_ref, acc):
        ph = pl.program_id(0)
        i = pl.program_id(1)

        @pl.when(jnp.logical_and(ph == 0, i == 0))
        def _():
            acc[...] = jnp.zeros_like(acc)

        @pl.when(ph == 0)
        def _():
            xb = x_ref[...]
            acc[0:1, :] += jnp.sum(xb, axis=0, keepdims=True)
            acc[1:2, :] += jnp.sum(xb * xb, axis=0, keepdims=True)

        @pl.when(ph == 1)
        def _():
            h_ref[...] = _bn_apply(x_ref[...], acc, g_ref[...], b_ref[...])

    return pl.pallas_call(
        body,
        grid=(2, GRID),
        in_specs=[
            pl.BlockSpec((BR, DF), lambda p, i: (i, 0)),
            pl.BlockSpec((1, DF), lambda p, i: (0, 0)),
            pl.BlockSpec((1, DF), lambda p, i: (0, 0)),
        ],
        out_specs=pl.BlockSpec((BR, DF), lambda p, i: (i, 0)),
        out_shape=jax.ShapeDtypeStruct((N, DF), jnp.float32),
        scratch_shapes=[pltpu.VMEM((2, DF), jnp.float32)],
    )(x, g, b)


def _rmm(h, w):
    def body(h_ref, w_ref, r_ref):
        r_ref[...] = _mm_t(h_ref[...], w_ref[...])

    return pl.pallas_call(
        body,
        grid=(GRID,),
        in_specs=[
            pl.BlockSpec((BR, DF), lambda i: (i, 0)),
            pl.BlockSpec((DF, DF), lambda i: (0, 0)),
        ],
        out_specs=pl.BlockSpec((BR, DF), lambda i: (i, 0)),
        out_shape=jax.ShapeDtypeStruct((N, DF), jnp.float32),
    )(h, w)


def _combine_block(a0_ref, a1_ref, c0_ref, c1_ref, r_ref, w_ref, bl_ref):
    cnt = c0_ref[:, 0:1] + c1_ref[:, 0:1]
    inv = 1.0 / jnp.maximum(cnt, 1.0)
    m = (a0_ref[...] + a1_ref[...]) * inv
    z = _mm_t(m, w_ref[...]) + bl_ref[...] + r_ref[...]
    return jnp.maximum(z, 0.0)


_MID_SPECS = [
    pl.BlockSpec((BR, DF), lambda p, i: (i, 0)),
    pl.BlockSpec((BR, DF), lambda p, i: (i, 0)),
    pl.BlockSpec((BR, CW), lambda p, i: (i, 0)),
    pl.BlockSpec((BR, CW), lambda p, i: (i, 0)),
    pl.BlockSpec((BR, DF), lambda p, i: (i, 0)),
    pl.BlockSpec((DF, DF), lambda p, i: (0, 0)),
    pl.BlockSpec((1, DF), lambda p, i: (0, 0)),
    pl.BlockSpec((1, DF), lambda p, i: (0, 0)),
    pl.BlockSpec((1, DF), lambda p, i: (0, 0)),
]


def _mid(a0, a1, c0, c1, r, wl, bl, g, b):
    def body(a0_ref, a1_ref, c0_ref, c1_ref, r_ref, w_ref, bl_ref,
             g_ref, b_ref, o_ref, hbuf, acc):
        ph = pl.program_id(0)
        i = pl.program_id(1)

        @pl.when(jnp.logical_and(ph == 0, i == 0))
        def _():
            acc[...] = jnp.zeros_like(acc)

        @pl.when(ph == 0)
        def _():
            h = _combine_block(a0_ref, a1_ref, c0_ref, c1_ref, r_ref,
                               w_ref, bl_ref)
            hbuf[pl.ds(i * BR, BR), :] = h
            acc[0:1, :] += jnp.sum(h, axis=0, keepdims=True)
            acc[1:2, :] += jnp.sum(h * h, axis=0, keepdims=True)

        @pl.when(ph == 1)
        def _():
            o_ref[...] = _bn_apply(hbuf[pl.ds(i * BR, BR), :], acc,
                                   g_ref[...], b_ref[...])

    return pl.pallas_call(
        body,
        grid=(2, GRID),
        in_specs=_MID_SPECS,
        out_specs=pl.BlockSpec((BR, DF), lambda p, i: (i, 0)),
        out_shape=jax.ShapeDtypeStruct((N, DF), jnp.float32),
        scratch_shapes=[pltpu.VMEM((N, DF), jnp.float32),
                        pltpu.VMEM((2, DF), jnp.float32)],
    )(a0, a1, c0, c1, r, wl, bl, g, b)


def _post(a0, a1, c0, c1, r, wl, bl, g, b, w1, b1, w2, b2, wo, bo):
    def body(a0_ref, a1_ref, c0_ref, c1_ref, r_ref, w_ref, bl_ref,
             g_ref, b_ref, w1_ref, b1_ref, w2_ref, b2_ref, wo_ref, bo_ref,
             o_ref, hbuf, acc):
        ph = pl.program_id(0)
        i = pl.program_id(1)

        @pl.when(jnp.logical_and(ph == 0, i == 0))
        def _():
            acc[...] = jnp.zeros_like(acc)

        @pl.when(ph == 0)
        def _():
            h = _combine_block(a0_ref, a1_ref, c0_ref, c1_ref, r_ref,
                               w_ref, bl_ref)
            hbuf[pl.ds(i * BR, BR), :] = h
            acc[0:1, :] += jnp.sum(h, axis=0, keepdims=True)
            acc[1:2, :] += jnp.sum(h * h, axis=0, keepdims=True)

        @pl.when(ph == 1)
        def _():
            hb = _bn_apply(hbuf[pl.ds(i * BR, BR), :], acc,
                           g_ref[...], b_ref[...])
            z1 = jnp.maximum(_mm_t(hb, w1_ref[...]) + b1_ref[...], 0.0)
            z2 = jnp.maximum(_mm_t(z1, w2_ref[...]) + b2_ref[...], 0.0)
            o = _mm_t(z2, wo_ref[...]) + bo_ref[...]
            mx = jnp.max(o, axis=1, keepdims=True)
            e = jnp.exp(o - mx)
            o_ref[...] = e / jnp.sum(e, axis=1, keepdims=True)

    return pl.pallas_call(
        body,
        grid=(2, GRID),
        in_specs=_MID_SPECS + [
            pl.BlockSpec((L1S, DF), lambda p, i: (0, 0)),
            pl.BlockSpec((1, L1S), lambda p, i: (0, 0)),
            pl.BlockSpec((L2S, L1S), lambda p, i: (0, 0)),
            pl.BlockSpec((1, L2S), lambda p, i: (0, 0)),
            pl.BlockSpec((OUTS, L2S), lambda p, i: (0, 0)),
            pl.BlockSpec((1, OUTS), lambda p, i: (0, 0)),
        ],
        out_specs=pl.BlockSpec((BR, OUTS), lambda p, i: (i, 0)),
        out_shape=jax.ShapeDtypeStruct((N, OUTS), jnp.float32),
        scratch_shapes=[pltpu.VMEM((N, DF), jnp.float32),
                        pltpu.VMEM((2, DF), jnp.float32)],
    )(a0, a1, c0, c1, r, wl, bl, g, b, w1, b1, w2, b2, wo, bo)


def kernel(x, edge_index, batch, gamma1, beta1, gamma2, beta2,
           Wl1, bl1, Wr1, Wl2, bl2, Wr2, W1, b1, W2, b2, Wo, bo):
    f32 = jnp.float32
    pad_src = jnp.mod(jnp.arange(EPAD - E, dtype=jnp.int32), CHUNK)
    src2d = jnp.concatenate(
        [edge_index[0], pad_src]).reshape(NCHP, CHUNK)
    pad_dst = N + jnp.mod(jnp.arange(EPAD - E, dtype=jnp.int32), CHUNK)
    dst2d = jnp.concatenate(
        [edge_index[1], pad_dst]).reshape(NCHP, CHUNK)
    zf = jnp.zeros((NROWS, DF), f32)
    zc = jnp.zeros((NROWS, CW), f32)
    ones16 = jnp.ones((CHUNK, CW), f32)
    g1 = gamma1.reshape(1, DF)
    be1 = beta1.reshape(1, DF)
    g2 = gamma2.reshape(1, DF)
    be2 = beta2.reshape(1, DF)
    bl1r = bl1.reshape(1, DF)
    bl2r = bl2.reshape(1, DF)
    b1r = b1.reshape(1, L1S)
    b2r = b2.reshape(1, L2S)
    bor = bo.reshape(1, OUTS)

    h0 = _pre(x, g1, be1)
    a0, a1, c0, c1 = _sc_agg_with_cnt(h0, src2d, dst2d, zf, zc, ones16)
    r0 = _rmm(h0, Wr1)
    h1b = _mid(a0, a1, c0, c1, r0, Wl1, bl1r, g2, be2)
    a0b, a1b = _sc_agg(h1b, src2d, dst2d, zf)
    r1 = _rmm(h1b, Wr2)
    return _post(a0b, a1b, c0, c1, r1, Wl2, bl2r, g2, be2,
                 W1, b1r, W2, b2r, Wo, bor)


# speedup vs baseline: 1.1759x; 1.1759x over previous
import functools

import jax
import jax.numpy as jnp
from jax import lax
from jax.experimental import pallas as pl
from jax.experimental.pallas import tpu as pltpu
from jax.experimental.pallas import tpu_sc as plsc

N = 10000
DF = 128
E = 320000
L1S = 128
L2S = 64
OUTS = 16
EPS = 1e-5

CHUNK = 128
NC = 2
NS = 16
NW = NC * NS
NCH_T = 80
NCHP = NW * NCH_T
EPAD = NCHP * CHUNK
NROWS = N + CHUNK
RPT = N // NS
CW = 16
NPAIR = NCH_T // 2

BR = 2000
GRID = N // BR


def _sc_mesh():
    return plsc.VectorSubcoreMesh(core_axis_name="c", subcore_axis_name="s")


_SC_PARAMS = pltpu.CompilerParams(use_tc_tiling_on_sc=False)


def _agg_body_common(h_hbm, src_hbm, dst_hbm, zf_hbm, agg_s,
                     isrc, idst, rows_a, rows_b, sem_a, sem_b,
                     wid, with_cnt=None):
    ones_v, cnt_s = with_cnt if with_cnt is not None else (None, None)
    start = wid * NCH_T

    def stage(pair, slot):
        pltpu.sync_copy(src_hbm.at[pl.ds(start + 2 * pair, 2)], isrc.at[slot])
        pltpu.sync_copy(dst_hbm.at[pl.ds(start + 2 * pair, 2)], idst.at[slot])

    def fire(slot, b, rows, sem):
        pltpu.async_copy(h_hbm.at[isrc.at[slot, b]], rows, sem)

    def drain_scatter(slot, b, rows, sem):
        pltpu.make_async_copy(zf_hbm.at[pl.ds(0, CHUNK)], rows, sem).wait()
        pltpu.sync_copy(rows, agg_s.at[idst.at[slot, b]], add=True)
        if ones_v is not None:
            pltpu.sync_copy(ones_v, cnt_s.at[idst.at[slot, b]], add=True)

    stage(0, 0)
    fire(0, 0, rows_a, sem_a)
    fire(0, 1, rows_b, sem_b)

    def step(t2, carry):
        p0 = 2 * t2
        p1 = 2 * t2 + 1

        stage(p1, 1)
        drain_scatter(0, 0, rows_a, sem_a)
        fire(1, 0, rows_a, sem_a)
        drain_scatter(0, 1, rows_b, sem_b)
        fire(1, 1, rows_b, sem_b)

        @pl.when(p1 + 1 < NPAIR)
        def _():
            stage(p1 + 1, 0)

        drain_scatter(1, 0, rows_a, sem_a)

        @pl.when(p1 + 1 < NPAIR)
        def _():
            fire(0, 0, rows_a, sem_a)

        drain_scatter(1, 1, rows_b, sem_b)

        @pl.when(p1 + 1 < NPAIR)
        def _():
            fire(0, 1, rows_b, sem_b)

        return carry

    lax.fori_loop(0, NPAIR // 2, step, 0)


def _sc_agg_with_cnt(h, src2d, dst2d, zf, zc, ones16):
    f32 = jnp.float32

    @functools.partial(
        pl.kernel,
        mesh=_sc_mesh(),
        out_type=[
            jax.ShapeDtypeStruct((N, DF), f32),
            jax.ShapeDtypeStruct((N, DF), f32),
            jax.ShapeDtypeStruct((N, CW), f32),
            jax.ShapeDtypeStruct((N, CW), f32),
        ],
        scratch_types=[
            pltpu.VMEM_SHARED((NROWS, DF), f32),
            pltpu.VMEM_SHARED((NROWS, CW), f32),
            pltpu.VMEM((2, 2, CHUNK), jnp.int32),
            pltpu.VMEM((2, 2, CHUNK), jnp.int32),
            pltpu.VMEM((CHUNK, DF), f32),
            pltpu.VMEM((CHUNK, DF), f32),
            pltpu.VMEM((CHUNK, CW), f32),
            pltpu.SemaphoreType.DMA,
            pltpu.SemaphoreType.DMA,
        ],
        compiler_params=_SC_PARAMS,
    )
    def k(h_hbm, src_hbm, dst_hbm, zf_hbm, zc_hbm, ones_hbm,
          a0_out, a1_out, c0_out, c1_out,
          agg_s, cnt_s, isrc, idst, rows_a, rows_b, ones_v, sem_a, sem_b):
        cid = lax.axis_index("c")
        sid = lax.axis_index("s")
        wid = sid * NC + cid
        r0 = sid * RPT
        pltpu.sync_copy(zf_hbm.at[pl.ds(r0, RPT)], agg_s.at[pl.ds(r0, RPT)])
        pltpu.sync_copy(zc_hbm.at[pl.ds(r0, RPT)], cnt_s.at[pl.ds(r0, RPT)])
        pltpu.sync_copy(ones_hbm, ones_v)

        @pl.when(sid == NS - 1)
        def _():
            pltpu.sync_copy(zf_hbm.at[pl.ds(N, NROWS - N)],
                            agg_s.at[pl.ds(N, NROWS - N)])
            pltpu.sync_copy(zc_hbm.at[pl.ds(N, NROWS - N)],
                            cnt_s.at[pl.ds(N, NROWS - N)])

        plsc.subcore_barrier()
        _agg_body_common(h_hbm, src_hbm, dst_hbm, zf_hbm, agg_s,
                         isrc, idst, rows_a, rows_b, sem_a, sem_b,
                         wid, with_cnt=(ones_v, cnt_s))
        plsc.subcore_barrier()

        @pl.when(cid == 0)
        def _():
            pltpu.sync_copy(agg_s.at[pl.ds(r0, RPT)], a0_out.at[pl.ds(r0, RPT)])
            pltpu.sync_copy(cnt_s.at[pl.ds(r0, RPT)], c0_out.at[pl.ds(r0, RPT)])

        @pl.when(cid == 1)
        def _():
            pltpu.sync_copy(agg_s.at[pl.ds(r0, RPT)], a1_out.at[pl.ds(r0, RPT)])
            pltpu.sync_copy(cnt_s.at[pl.ds(r0, RPT)], c1_out.at[pl.ds(r0, RPT)])

    return k(h, src2d, dst2d, zf, zc, ones16)


def _sc_agg(h, src2d, dst2d, zf):
    f32 = jnp.float32

    @functools.partial(
        pl.kernel,
        mesh=_sc_mesh(),
        out_type=[
            jax.ShapeDtypeStruct((N, DF), f32),
            jax.ShapeDtypeStruct((N, DF), f32),
        ],
        scratch_types=[
            pltpu.VMEM_SHARED((NROWS, DF), f32),
            pltpu.VMEM((2, 2, CHUNK), jnp.int32),
            pltpu.VMEM((2, 2, CHUNK), jnp.int32),
            pltpu.VMEM((CHUNK, DF), f32),
            pltpu.VMEM((CHUNK, DF), f32),
            pltpu.SemaphoreType.DMA,
            pltpu.SemaphoreType.DMA,
        ],
        compiler_params=_SC_PARAMS,
    )
    def k(h_hbm, src_hbm, dst_hbm, zf_hbm,
          a0_out, a1_out,
          agg_s, isrc, idst, rows_a, rows_b, sem_a, sem_b):
        cid = lax.axis_index("c")
        sid = lax.axis_index("s")
        wid = sid * NC + cid
        r0 = sid * RPT
        pltpu.sync_copy(zf_hbm.at[pl.ds(r0, RPT)], agg_s.at[pl.ds(r0, RPT)])

        @pl.when(sid == NS - 1)
        def _():
            pltpu.sync_copy(zf_hbm.at[pl.ds(N, NROWS - N)],
                            agg_s.at[pl.ds(N, NROWS - N)])

        plsc.subcore_barrier()
        _agg_body_common(h_hbm, src_hbm, dst_hbm, zf_hbm, agg_s,
                         isrc, idst, rows_a, rows_b, sem_a, sem_b, wid)
        plsc.subcore_barrier()

        @pl.when(cid == 0)
        def _():
            pltpu.sync_copy(agg_s.at[pl.ds(r0, RPT)], a0_out.at[pl.ds(r0, RPT)])

        @pl.when(cid == 1)
        def _():
            pltpu.sync_copy(agg_s.at[pl.ds(r0, RPT)], a1_out.at[pl.ds(r0, RPT)])

    return k(h, src2d, dst2d, zf)


def _mm_t(a, w):
    return lax.dot_general(a, w, (((1,), (1,)), ((), ())),
                           preferred_element_type=jnp.float32)


def _bn_apply(x, acc, g, b):
    mean = acc[0:1, :] * (1.0 / N)
    var = acc[1:2, :] * (1.0 / N) - mean * mean
    return (x - mean) * lax.rsqrt(var + EPS) * g + b


def _pre(x, g, b):
    def body(x_ref, g_ref, b_ref, h_ref, acc):
        ph = pl.program_id(0)
        i = pl.program_id(1)

        @pl.when(jnp.logical_and(ph == 0, i == 0))
        def _():
            acc[...] = jnp.zeros_like(acc)

        @pl.when(ph == 0)
        def _():
            xb = x_ref[...]
            acc[0:1, :] += jnp.sum(xb, axis=0, keepdims=True)
            acc[1:2, :] += jnp.sum(xb * xb, axis=0, keepdims=True)

        @pl.when(ph == 1)
        def _():
            h_ref[...] = _bn_apply(x_ref[...], acc, g_ref[...], b_ref[...])

    return pl.pallas_call(
        body,
        grid=(2, GRID),
        in_specs=[
            pl.BlockSpec((BR, DF), lambda p, i: (i, 0)),
            pl.BlockSpec((1, DF), lambda p, i: (0, 0)),
            pl.BlockSpec((1, DF), lambda p, i: (0, 0)),
        ],
        out_specs=pl.BlockSpec((BR, DF), lambda p, i: (i, 0)),
        out_shape=jax.ShapeDtypeStruct((N, DF), jnp.float32),
        scratch_shapes=[pltpu.VMEM((2, DF), jnp.float32)],
    )(x, g, b)


def _rmm(h, w):
    def body(h_ref, w_ref, r_ref):
        r_ref[...] = _mm_t(h_ref[...], w_ref[...])

    return pl.pallas_call(
        body,
        grid=(GRID,),
        in_specs=[
            pl.BlockSpec((BR, DF), lambda i: (i, 0)),
            pl.BlockSpec((DF, DF), lambda i: (0, 0)),
        ],
        out_specs=pl.BlockSpec((BR, DF), lambda i: (i, 0)),
        out_shape=jax.ShapeDtypeStruct((N, DF), jnp.float32),
    )(h, w)


def _combine_block(a0_ref, a1_ref, c0_ref, c1_ref, r_ref, w_ref, bl_ref):
    cnt = c0_ref[:, 0:1] + c1_ref[:, 0:1]
    inv = 1.0 / jnp.maximum(cnt, 1.0)
    m = (a0_ref[...] + a1_ref[...]) * inv
    z = _mm_t(m, w_ref[...]) + bl_ref[...] + r_ref[...]
    return jnp.maximum(z, 0.0)


_MID_SPECS = [
    pl.BlockSpec((BR, DF), lambda p, i: (i, 0)),
    pl.BlockSpec((BR, DF), lambda p, i: (i, 0)),
    pl.BlockSpec((BR, CW), lambda p, i: (i, 0)),
    pl.BlockSpec((BR, CW), lambda p, i: (i, 0)),
    pl.BlockSpec((BR, DF), lambda p, i: (i, 0)),
    pl.BlockSpec((DF, DF), lambda p, i: (0, 0)),
    pl.BlockSpec((1, DF), lambda p, i: (0, 0)),
    pl.BlockSpec((1, DF), lambda p, i: (0, 0)),
    pl.BlockSpec((1, DF), lambda p, i: (0, 0)),
]


def _mid(a0, a1, c0, c1, r, wl, bl, g, b):
    def body(a0_ref, a1_ref, c0_ref, c1_ref, r_ref, w_ref, bl_ref,
             g_ref, b_ref, o_ref, hbuf, acc):
        ph = pl.program_id(0)
        i = pl.program_id(1)

        @pl.when(jnp.logical_and(ph == 0, i == 0))
        def _():
            acc[...] = jnp.zeros_like(acc)

        @pl.when(ph == 0)
        def _():
            h = _combine_block(a0_ref, a1_ref, c0_ref, c1_ref, r_ref,
                               w_ref, bl_ref)
            hbuf[pl.ds(i * BR, BR), :] = h
            acc[0:1, :] += jnp.sum(h, axis=0, keepdims=True)
            acc[1:2, :] += jnp.sum(h * h, axis=0, keepdims=True)

        @pl.when(ph == 1)
        def _():
            o_ref[...] = _bn_apply(hbuf[pl.ds(i * BR, BR), :], acc,
                                   g_ref[...], b_ref[...])

    return pl.pallas_call(
        body,
        grid=(2, GRID),
        in_specs=_MID_SPECS,
        out_specs=pl.BlockSpec((BR, DF), lambda p, i: (i, 0)),
        out_shape=jax.ShapeDtypeStruct((N, DF), jnp.float32),
        scratch_shapes=[pltpu.VMEM((N, DF), jnp.float32),
                        pltpu.VMEM((2, DF), jnp.float32)],
    )(a0, a1, c0, c1, r, wl, bl, g, b)


def _post(a0, a1, c0, c1, r, wl, bl, g, b, w1, b1, w2, b2, wo, bo):
    def body(a0_ref, a1_ref, c0_ref, c1_ref, r_ref, w_ref, bl_ref,
             g_ref, b_ref, w1_ref, b1_ref, w2_ref, b2_ref, wo_ref, bo_ref,
             o_ref, hbuf, acc):
        ph = pl.program_id(0)
        i = pl.program_id(1)

        @pl.when(jnp.logical_and(ph == 0, i == 0))
        def _():
            acc[...] = jnp.zeros_like(acc)

        @pl.when(ph == 0)
        def _():
            h = _combine_block(a0_ref, a1_ref, c0_ref, c1_ref, r_ref,
                               w_ref, bl_ref)
            hbuf[pl.ds(i * BR, BR), :] = h
            acc[0:1, :] += jnp.sum(h, axis=0, keepdims=True)
            acc[1:2, :] += jnp.sum(h * h, axis=0, keepdims=True)

        @pl.when(ph == 1)
        def _():
            hb = _bn_apply(hbuf[pl.ds(i * BR, BR), :], acc,
                           g_ref[...], b_ref[...])
            z1 = jnp.maximum(_mm_t(hb, w1_ref[...]) + b1_ref[...], 0.0)
            z2 = jnp.maximum(_mm_t(z1, w2_ref[...]) + b2_ref[...], 0.0)
            o = _mm_t(z2, wo_ref[...]) + bo_ref[...]
            mx = jnp.max(o, axis=1, keepdims=True)
            e = jnp.exp(o - mx)
            o_ref[...] = e / jnp.sum(e, axis=1, keepdims=True)

    return pl.pallas_call(
        body,
        grid=(2, GRID),
        in_specs=_MID_SPECS + [
            pl.BlockSpec((L1S, DF), lambda p, i: (0, 0)),
            pl.BlockSpec((1, L1S), lambda p, i: (0, 0)),
            pl.BlockSpec((L2S, L1S), lambda p, i: (0, 0)),
            pl.BlockSpec((1, L2S), lambda p, i: (0, 0)),
            pl.BlockSpec((OUTS, L2S), lambda p, i: (0, 0)),
            pl.BlockSpec((1, OUTS), lambda p, i: (0, 0)),
        ],
        out_specs=pl.BlockSpec((BR, OUTS), lambda p, i: (i, 0)),
        out_shape=jax.ShapeDtypeStruct((N, OUTS), jnp.float32),
        scratch_shapes=[pltpu.VMEM((N, DF), jnp.float32),
                        pltpu.VMEM((2, DF), jnp.float32)],
    )(a0, a1, c0, c1, r, wl, bl, g, b, w1, b1, w2, b2, wo, bo)


def kernel(x, edge_index, batch, gamma1, beta1, gamma2, beta2,
           Wl1, bl1, Wr1, Wl2, bl2, Wr2, W1, b1, W2, b2, Wo, bo):
    f32 = jnp.float32
    pad_src = jnp.mod(jnp.arange(EPAD - E, dtype=jnp.int32), CHUNK)
    src2d = jnp.concatenate(
        [edge_index[0], pad_src]).reshape(NCHP, CHUNK)
    pad_dst = N + jnp.mod(jnp.arange(EPAD - E, dtype=jnp.int32), CHUNK)
    dst2d = jnp.concatenate(
        [edge_index[1], pad_dst]).reshape(NCHP, CHUNK)
    zf = jnp.zeros((NROWS, DF), f32)
    zc = jnp.zeros((NROWS, CW), f32)
    ones16 = jnp.ones((CHUNK, CW), f32)
    g1 = gamma1.reshape(1, DF)
    be1 = beta1.reshape(1, DF)
    g2 = gamma2.reshape(1, DF)
    be2 = beta2.reshape(1, DF)
    bl1r = bl1.reshape(1, DF)
    bl2r = bl2.reshape(1, DF)
    b1r = b1.reshape(1, L1S)
    b2r = b2.reshape(1, L2S)
    bor = bo.reshape(1, OUTS)

    h0 = _pre(x, g1, be1)
    a0, a1, c0, c1 = _sc_agg_with_cnt(h0, src2d, dst2d, zf, zc, ones16)
    r0 = _rmm(h0, Wr1)
    h1b = _mid(a0, a1, c0, c1, r0, Wl1, bl1r, g2, be2)
    a0b, a1b = _sc_agg(h1b, src2d, dst2d, zf)
    r1 = _rmm(h1b, Wr2)
    return _post(a0b, a1b, c0, c1, r1, Wl2, bl2r, g2, be2,
                 W1, b1r, W2, b2r, Wo, bor)
